# baseline (speedup 1.0000x reference)
import functools

import jax
import jax.numpy as jnp
from jax import lax
from jax.experimental import pallas as pl
from jax.experimental.pallas import tpu as pltpu
from jax.experimental.pallas import tpu_sc as plsc

_H = 32
_HH = 16
_NSC = 2
_NT = 16
_EC = 400
_P = 8
_R8 = 1024


def _cdiv(a, b):
  return (a + b - 1) // b


_SC_PARAMS = pltpu.CompilerParams(use_tc_tiling_on_sc=False)


@functools.cache
def _deg_kernel(B, N, E):
  assert B == _NSC
  npad = _cdiv(N, 256) * 256
  stripe = npad // _NT
  ept = E // _NT
  nchunk = ept // _EC
  assert ept % _EC == 0
  mesh = plsc.VectorSubcoreMesh(core_axis_name="c", subcore_axis_name="s")

  @functools.partial(
      pl.kernel,
      out_type=jax.ShapeDtypeStruct((B, npad), jnp.float32),
      mesh=mesh,
      compiler_params=_SC_PARAMS,
      scratch_types=[
          pltpu.VMEM((_EC,), jnp.int32),
          pltpu.VMEM((_EC,), jnp.float32),
          pltpu.VMEM_SHARED((npad,), jnp.float32),
      ],
  )
  def deg(ei_hbm, ones_hbm, zeros_hbm, degp_hbm, cbuf, obuf, acc):
    c = lax.axis_index("c")
    s = lax.axis_index("s")
    r0 = pl.multiple_of(s * stripe, stripe)
    pltpu.sync_copy(zeros_hbm.at[pl.ds(r0, stripe)], acc.at[pl.ds(r0, stripe)])
    pltpu.sync_copy(ones_hbm, obuf)
    plsc.subcore_barrier()

    def body(j, carry):
      base = pl.multiple_of(s * ept + j * _EC, _EC)
      pltpu.sync_copy(ei_hbm.at[c, 1, pl.ds(base, _EC)], cbuf)
      pltpu.sync_copy(obuf, acc.at[cbuf], add=True)
      return carry

    lax.fori_loop(0, nchunk, body, 0)
    plsc.subcore_barrier()
    pltpu.sync_copy(acc.at[pl.ds(r0, stripe)],
                    degp_hbm.at[c].at[pl.ds(r0, stripe)])

  return deg


@functools.cache
def _prop_kernel(B, N, E):
  assert N % _NT == 0
  stripe = N // _NT
  ept = E // _NT
  nchunk = ept // _EC
  assert ept % _EC == 0
  mesh = plsc.VectorSubcoreMesh(core_axis_name="c", subcore_axis_name="s")

  ns = 4

  @functools.partial(
      pl.kernel,
      out_type=jax.ShapeDtypeStruct((_NSC, B, N, _HH), jnp.float32),
      mesh=mesh,
      compiler_params=_SC_PARAMS,
      scratch_types=[
          pltpu.VMEM((ns, _EC), jnp.int32),
          pltpu.VMEM((ns, _EC), jnp.int32),
          pltpu.VMEM((ns, _EC, _HH), jnp.float32),
          pltpu.VMEM_SHARED((N, _HH), jnp.float32),
          pltpu.SemaphoreType.DMA((ns,)),
          pltpu.SemaphoreType.DMA((ns,)),
          pltpu.SemaphoreType.DMA((ns,)),
      ],
  )
  def prop(ei_hbm, z_hbm, zeros_hbm, a_hbm, rbuf, cbuf, dbuf, acc, isem,
           gsem, ssem):
    c = lax.axis_index("c")
    s = lax.axis_index("s")
    r0 = pl.multiple_of(s * stripe, stripe)
    for g in range(B):
      pltpu.sync_copy(zeros_hbm.at[pl.ds(r0, stripe)],
                      acc.at[pl.ds(r0, stripe)])
      plsc.subcore_barrier()

      il = [None] * ns
      gat = [None] * ns
      scat = [None] * ns

      def idxload(j):
        k = j % ns
        base = pl.multiple_of(s * ept + j * _EC, _EC)
        d0 = pltpu.async_copy(ei_hbm.at[g, 0, pl.ds(base, _EC)], rbuf.at[k],
                              isem.at[k])
        d1 = pltpu.async_copy(ei_hbm.at[g, 1, pl.ds(base, _EC)], cbuf.at[k],
                              isem.at[k])
        il[k] = (d0, d1)

      def gather(j):
        k = j % ns
        d0, d1 = il[k]
        d0.wait()
        d1.wait()
        il[k] = None
        gat[k] = pltpu.async_copy(z_hbm.at[c, g].at[rbuf.at[k]], dbuf.at[k],
                                  gsem.at[k])

      def drain_scat(k):
        if scat[k] is not None:
          scat[k].wait()
          scat[k] = None

      idxload(0)
      if nchunk > 1:
        idxload(1)
      gather(0)
      for j in range(nchunk):
        if j + 2 < nchunk:
          drain_scat((j + 2) % ns)
          idxload(j + 2)
        if j + 1 < nchunk:
          drain_scat((j + 1) % ns)
          gather(j + 1)
        k = j % ns
        gat[k].wait()
        scat[k] = pltpu.async_copy(dbuf.at[k], acc.at[cbuf.at[k]], ssem.at[k],
                                   add=True)
      for k in range(ns):
        drain_scat(k)
      plsc.subcore_barrier()
      pltpu.sync_copy(acc.at[pl.ds(r0, stripe)],
                      a_hbm.at[c, g].at[pl.ds(r0, stripe)])

  return prop


def _mm(x, w_ref):
  return jnp.dot(x, w_ref[...], preferred_element_type=jnp.float32)


def _pre_body(deg8_ref, h0p_ref, wl_ref, wh_ref, q_ref, dis_ref, z_ref):
  deg8 = deg8_ref[...]
  dis8 = jnp.where(deg8 > 0.0,
                   lax.rsqrt(jnp.where(deg8 > 0.0, deg8, 1.0)), 0.0)
  dis = jnp.dot(dis8, q_ref[...], preferred_element_type=jnp.float32)
  dis_ref[...] = dis
  h0p = h0p_ref[...]
  z_ref[0] = dis * _mm(h0p, wl_ref)
  z_ref[1] = dis * _mm(h0p, wh_ref)


def _mid0_body(h0p_ref, a_ref, dis_ref, wl_ref, wh_ref, bl_ref, bh_ref,
               vll_ref, vhl_ref, vlh_ref, vhh_ref, hn_ref, zn_ref):
  dis = dis_ref[...]
  h0p = h0p_ref[...]
  hnl = jnp.tanh(_mm(h0p, wl_ref) + dis * a_ref[0] + bl_ref[...][None, :])
  hnh = jnp.tanh(_mm(h0p, wh_ref) + dis * a_ref[1] + bh_ref[...][None, :])
  hn_ref[0] = hnl
  hn_ref[1] = hnh
  zn_ref[0] = dis * (_mm(hnl, vll_ref) + _mm(hnh, vhl_ref))
  zn_ref[1] = dis * (_mm(hnl, vlh_ref) + _mm(hnh, vhh_ref))


def _mid_body(h_ref, a_ref, dis_ref, wll_ref, whl_ref, wlh_ref,
              whh_ref, bl_ref, bh_ref, vll_ref, vhl_ref, vlh_ref, vhh_ref,
              hn_ref, zn_ref):
  dis = dis_ref[...]
  hl, hh = h_ref[0], h_ref[1]
  hnl = jnp.tanh(_mm(hl, wll_ref) + _mm(hh, whl_ref) + dis * a_ref[0]
                 + bl_ref[...][None, :])
  hnh = jnp.tanh(_mm(hl, wlh_ref) + _mm(hh, whh_ref) + dis * a_ref[1]
                 + bh_ref[...][None, :])
  hn_ref[0] = hnl
  hn_ref[1] = hnh
  zn_ref[0] = dis * (_mm(hnl, vll_ref) + _mm(hnh, vhl_ref))
  zn_ref[1] = dis * (_mm(hnl, vlh_ref) + _mm(hnh, vhh_ref))


def _fin_body(h_ref, a_ref, dis_ref, wll_ref, whl_ref, wlh_ref,
              whh_ref, bl_ref, bh_ref, rl_ref, rh_ref, rb_ref, sl_ref,
              sh_ref, hn_ref, y_ref):
  dis = dis_ref[...]
  hl, hh = h_ref[0], h_ref[1]
  hnl = jnp.tanh(_mm(hl, wll_ref) + _mm(hh, whl_ref) + dis * a_ref[0]
                 + bl_ref[...][None, :])
  hnh = jnp.tanh(_mm(hl, wlh_ref) + _mm(hh, whh_ref) + dis * a_ref[1]
                 + bh_ref[...][None, :])
  hn_ref[...] = _mm(hnl, sl_ref) + _mm(hnh, sh_ref)
  y_ref[...] = _mm(hnl, rl_ref) + _mm(hnh, rh_ref) + rb_ref[...]


def _full(shape):
  return pl.BlockSpec(shape, lambda i: (0,) * len(shape))


def _rows(din):
  return pl.BlockSpec((_R8, din), lambda i: (i, 0))


_PAIR = pl.BlockSpec((_NSC, _R8, 128), lambda i: (0, i, 0))


def _tc_pre(T8, deg8, h0p, wl, wh, q):
  grid = (_cdiv(T8, _R8),)
  return pl.pallas_call(
      _pre_body,
      grid=grid,
      in_specs=[_rows(_P), _rows(h0p.shape[1]), _full(wl.shape),
                _full(wh.shape), _full(q.shape)],
      out_specs=[_rows(128), _PAIR],
      out_shape=[
          jax.ShapeDtypeStruct((T8, 128), jnp.float32),
          jax.ShapeDtypeStruct((_NSC, T8, 128), jnp.float32),
      ],
  )(deg8, h0p, wl, wh, q)


def _tc_mid0(T8, h0p, a, dis, wl, wh, bs, vs):
  grid = (_cdiv(T8, _R8),)
  return pl.pallas_call(
      _mid0_body,
      grid=grid,
      in_specs=[
          _rows(h0p.shape[1]), _PAIR, _rows(128),
          _full(wl.shape), _full(wh.shape),
          *[_full(b.shape) for b in bs],
          *[_full(v.shape) for v in vs],
      ],
      out_specs=[_PAIR, _PAIR],
      out_shape=[
          jax.ShapeDtypeStruct((_NSC, T8, 128), jnp.float32),
          jax.ShapeDtypeStruct((_NSC, T8, 128), jnp.float32),
      ],
  )(h0p, a, dis, wl, wh, *bs, *vs)


def _tc_mid(T8, h, a, dis, ws, bs, vs):
  grid = (_cdiv(T8, _R8),)
  return pl.pallas_call(
      _mid_body,
      grid=grid,
      in_specs=[
          _PAIR, _PAIR, _rows(128),
          *[_full(w.shape) for w in ws],
          *[_full(b.shape) for b in bs],
          *[_full(v.shape) for v in vs],
      ],
      out_specs=[_PAIR, _PAIR],
      out_shape=[
          jax.ShapeDtypeStruct((_NSC, T8, 128), jnp.float32),
          jax.ShapeDtypeStruct((_NSC, T8, 128), jnp.float32),
      ],
  )(h, a, dis, *ws, *bs, *vs)


def _tc_fin(T8, h, a, dis, ws, bs, rs, rb, sl, sh):
  grid = (_cdiv(T8, _R8),)
  return pl.pallas_call(
      _fin_body,
      grid=grid,
      in_specs=[
          _PAIR, _PAIR, _rows(128),
          *[_full(w.shape) for w in ws],
          *[_full(b.shape) for b in bs],
          *[_full(r.shape) for r in rs],
          _full(rb.shape), _full(sl.shape), _full(sh.shape),
      ],
      out_specs=[_rows(256), _rows(_P)],
      out_shape=[
          jax.ShapeDtypeStruct((T8, 256), jnp.float32),
          jax.ShapeDtypeStruct((T8, _P), jnp.float32),
      ],
  )(h, a, dis, *ws, *bs, *rs, rb, sl, sh)


def kernel(X, edge_index, conv0_w0, conv0_w1, conv0_b, conv1_w0, conv1_w1,
           conv1_b, conv2_w0, conv2_w1, conv2_b, reg_w, reg_b):
  B, W, N = X.shape
  E = edge_index.shape[2]
  T = B * N
  T8 = T // _P
  npad = _cdiv(N, 256) * 256
  eye8 = jnp.eye(_P, dtype=jnp.float32)

  def k8(w):
    return jnp.kron(eye8, w)

  def wsplit(w):
    return [k8(w[:_HH, :_HH]), k8(w[_HH:, :_HH]),
            k8(w[:_HH, _HH:]), k8(w[_HH:, _HH:])]

  h0p = X.reshape(T8, _P * W)
  q = jnp.kron(eye8, jnp.ones((1, _HH), jnp.float32))
  ones_e = jnp.ones((_EC,), jnp.float32)
  zeros_d = jnp.zeros((npad,), jnp.float32)
  zeros_f = jnp.zeros((N, _HH), jnp.float32)

  degp = _deg_kernel(B, N, E)(edge_index, ones_e, zeros_d)
  deg8 = degp[:, :N].reshape(T8, _P)
  dis, z = _tc_pre(T8, deg8, h0p, k8(conv0_w1[:, :_HH]),
                   k8(conv0_w1[:, _HH:]), q)
  prop = _prop_kernel(B, N, E)

  def _prop(zp):
    a = prop(edge_index, zp.reshape(_NSC, B, N, _HH), zeros_f)
    return a.reshape(_NSC, T8, 128)

  def bsplit(b):
    return [jnp.tile(b[:_HH], _P), jnp.tile(b[_HH:], _P)]

  a = _prop(z)
  h1, z = _tc_mid0(T8, h0p, a, dis, k8(conv0_w0[:, :_HH]),
                   k8(conv0_w0[:, _HH:]), bsplit(conv0_b),
                   wsplit(conv1_w1))
  a = _prop(z)
  h2, z = _tc_mid(T8, h1, a, dis, wsplit(conv1_w0), bsplit(conv1_b),
                  wsplit(conv2_w1))
  a = _prop(z)
  sl = k8(jnp.concatenate([jnp.eye(_HH, dtype=jnp.float32),
                           jnp.zeros((_HH, _HH), jnp.float32)], axis=1))
  sh = k8(jnp.concatenate([jnp.zeros((_HH, _HH), jnp.float32),
                           jnp.eye(_HH, dtype=jnp.float32)], axis=1))
  h3, y = _tc_fin(T8, h2, a, dis, wsplit(conv2_w0), bsplit(conv2_b),
                  [k8(reg_w[:_HH]), k8(reg_w[_HH:])], reg_b, sl, sh)
  return (y.reshape(B, N), h3.reshape(T, _H))


# speedup vs baseline: 1.1954x; 1.1954x over previous
import functools

import jax
import jax.numpy as jnp
from jax import lax
from jax.experimental import pallas as pl
from jax.experimental.pallas import tpu as pltpu
from jax.experimental.pallas import tpu_sc as plsc

_H = 32
_HH = 16
_NSC = 2
_NT = 16
_EC = 2000
_P = 8
_R8 = 1024


def _cdiv(a, b):
  return (a + b - 1) // b


_SC_PARAMS = pltpu.CompilerParams(use_tc_tiling_on_sc=False)


@functools.cache
def _deg_kernel(B, N, E):
  assert B == _NSC
  npad = _cdiv(N, 256) * 256
  stripe = npad // _NT
  ept = E // _NT
  nchunk = ept // _EC
  assert ept % _EC == 0
  mesh = plsc.VectorSubcoreMesh(core_axis_name="c", subcore_axis_name="s")

  @functools.partial(
      pl.kernel,
      out_type=jax.ShapeDtypeStruct((B, npad), jnp.float32),
      mesh=mesh,
      compiler_params=_SC_PARAMS,
      scratch_types=[
          pltpu.VMEM((_EC,), jnp.int32),
          pltpu.VMEM((_EC,), jnp.float32),
          pltpu.VMEM_SHARED((npad,), jnp.float32),
      ],
  )
  def deg(ei_hbm, ones_hbm, zeros_hbm, degp_hbm, cbuf, obuf, acc):
    c = lax.axis_index("c")
    s = lax.axis_index("s")
    r0 = pl.multiple_of(s * stripe, stripe)
    pltpu.sync_copy(zeros_hbm.at[pl.ds(r0, stripe)], acc.at[pl.ds(r0, stripe)])
    pltpu.sync_copy(ones_hbm, obuf)
    plsc.subcore_barrier()

    def body(j, carry):
      base = pl.multiple_of(s * ept + j * _EC, _EC)
      pltpu.sync_copy(ei_hbm.at[c, 1, pl.ds(base, _EC)], cbuf)
      pltpu.sync_copy(obuf, acc.at[cbuf], add=True)
      return carry

    lax.fori_loop(0, nchunk, body, 0)
    plsc.subcore_barrier()
    pltpu.sync_copy(acc.at[pl.ds(r0, stripe)],
                    degp_hbm.at[c].at[pl.ds(r0, stripe)])

  return deg


@functools.cache
def _prop_kernel(B, N, E):
  assert N % _NT == 0
  stripe = N // _NT
  ept = E // _NT
  nchunk = ept // _EC
  assert ept % _EC == 0
  mesh = plsc.VectorSubcoreMesh(core_axis_name="c", subcore_axis_name="s")

  ni = 4
  nd = 2

  @functools.partial(
      pl.kernel,
      out_type=jax.ShapeDtypeStruct((_NSC, B, N, _HH), jnp.float32),
      mesh=mesh,
      compiler_params=_SC_PARAMS,
      scratch_types=[
          pltpu.VMEM((ni, _EC), jnp.int32),
          pltpu.VMEM((ni, _EC), jnp.int32),
          pltpu.VMEM((nd, _EC, _HH), jnp.float32),
          pltpu.VMEM_SHARED((N, _HH), jnp.float32),
          pltpu.SemaphoreType.DMA((ni,)),
          pltpu.SemaphoreType.DMA((nd,)),
          pltpu.SemaphoreType.DMA((ni,)),
      ],
  )
  def prop(ei_hbm, z_hbm, zeros_hbm, a_hbm, rbuf, cbuf, dbuf, acc, isem,
           gsem, ssem):
    c = lax.axis_index("c")
    s = lax.axis_index("s")
    r0 = pl.multiple_of(s * stripe, stripe)
    for g in range(B):
      pltpu.sync_copy(zeros_hbm.at[pl.ds(r0, stripe)],
                      acc.at[pl.ds(r0, stripe)])
      plsc.subcore_barrier()

      il = [None] * ni
      gat = [None] * nd
      scat = [None] * ni

      def idxload(j):
        k = j % ni
        base = pl.multiple_of(s * ept + j * _EC, _EC)
        d0 = pltpu.async_copy(ei_hbm.at[g, 0, pl.ds(base, _EC)], rbuf.at[k],
                              isem.at[k])
        d1 = pltpu.async_copy(ei_hbm.at[g, 1, pl.ds(base, _EC)], cbuf.at[k],
                              isem.at[k])
        il[k] = (d0, d1)

      def gather(j):
        k = j % ni
        kd = j % nd
        d0, d1 = il[k]
        d0.wait()
        d1.wait()
        il[k] = None
        gat[kd] = pltpu.async_copy(z_hbm.at[c, g].at[rbuf.at[k]], dbuf.at[kd],
                                   gsem.at[kd])

      def drain_scat(k):
        if scat[k] is not None:
          scat[k].wait()
          scat[k] = None

      idxload(0)
      if nchunk > 1:
        idxload(1)
      gather(0)
      for j in range(nchunk):
        if j + 2 < nchunk:
          drain_scat((j + 2) % ni)
          idxload(j + 2)
        if j + 1 < nchunk:
          drain_scat((j + 3) % ni)
          gather(j + 1)
        gat[j % nd].wait()
        scat[j % ni] = pltpu.async_copy(dbuf.at[j % nd],
                                        acc.at[cbuf.at[j % ni]],
                                        ssem.at[j % ni], add=True)
      for k in range(ni):
        drain_scat(k)
      plsc.subcore_barrier()
      pltpu.sync_copy(acc.at[pl.ds(r0, stripe)],
                      a_hbm.at[c, g].at[pl.ds(r0, stripe)])

  return prop


def _mm(x, w_ref):
  return jnp.dot(x, w_ref[...], preferred_element_type=jnp.float32)


def _pre_body(deg8_ref, h0p_ref, wl_ref, wh_ref, q_ref, dis_ref, z_ref):
  deg8 = deg8_ref[...]
  dis8 = jnp.where(deg8 > 0.0,
                   lax.rsqrt(jnp.where(deg8 > 0.0, deg8, 1.0)), 0.0)
  dis = jnp.dot(dis8, q_ref[...], preferred_element_type=jnp.float32)
  dis_ref[...] = dis
  h0p = h0p_ref[...]
  z_ref[0] = dis * _mm(h0p, wl_ref)
  z_ref[1] = dis * _mm(h0p, wh_ref)


def _mid0_body(h0p_ref, a_ref, dis_ref, wl_ref, wh_ref, bl_ref, bh_ref,
               vll_ref, vhl_ref, vlh_ref, vhh_ref, hn_ref, zn_ref):
  dis = dis_ref[...]
  h0p = h0p_ref[...]
  hnl = jnp.tanh(_mm(h0p, wl_ref) + dis * a_ref[0] + bl_ref[...][None, :])
  hnh = jnp.tanh(_mm(h0p, wh_ref) + dis * a_ref[1] + bh_ref[...][None, :])
  hn_ref[0] = hnl
  hn_ref[1] = hnh
  zn_ref[0] = dis * (_mm(hnl, vll_ref) + _mm(hnh, vhl_ref))
  zn_ref[1] = dis * (_mm(hnl, vlh_ref) + _mm(hnh, vhh_ref))


def _mid_body(h_ref, a_ref, dis_ref, wll_ref, whl_ref, wlh_ref,
              whh_ref, bl_ref, bh_ref, vll_ref, vhl_ref, vlh_ref, vhh_ref,
              hn_ref, zn_ref):
  dis = dis_ref[...]
  hl, hh = h_ref[0], h_ref[1]
  hnl = jnp.tanh(_mm(hl, wll_ref) + _mm(hh, whl_ref) + dis * a_ref[0]
                 + bl_ref[...][None, :])
  hnh = jnp.tanh(_mm(hl, wlh_ref) + _mm(hh, whh_ref) + dis * a_ref[1]
                 + bh_ref[...][None, :])
  hn_ref[0] = hnl
  hn_ref[1] = hnh
  zn_ref[0] = dis * (_mm(hnl, vll_ref) + _mm(hnh, vhl_ref))
  zn_ref[1] = dis * (_mm(hnl, vlh_ref) + _mm(hnh, vhh_ref))


def _fin_body(h_ref, a_ref, dis_ref, wll_ref, whl_ref, wlh_ref,
              whh_ref, bl_ref, bh_ref, rl_ref, rh_ref, rb_ref, sl_ref,
              sh_ref, hn_ref, y_ref):
  dis = dis_ref[...]
  hl, hh = h_ref[0], h_ref[1]
  hnl = jnp.tanh(_mm(hl, wll_ref) + _mm(hh, whl_ref) + dis * a_ref[0]
                 + bl_ref[...][None, :])
  hnh = jnp.tanh(_mm(hl, wlh_ref) + _mm(hh, whh_ref) + dis * a_ref[1]
                 + bh_ref[...][None, :])
  hn_ref[...] = _mm(hnl, sl_ref) + _mm(hnh, sh_ref)
  y_ref[...] = _mm(hnl, rl_ref) + _mm(hnh, rh_ref) + rb_ref[...]


def _full(shape):
  return pl.BlockSpec(shape, lambda i: (0,) * len(shape))


def _rows(din):
  return pl.BlockSpec((_R8, din), lambda i: (i, 0))


_PAIR = pl.BlockSpec((_NSC, _R8, 128), lambda i: (0, i, 0))


def _tc_pre(T8, deg8, h0p, wl, wh, q):
  grid = (_cdiv(T8, _R8),)
  return pl.pallas_call(
      _pre_body,
      grid=grid,
      in_specs=[_rows(_P), _rows(h0p.shape[1]), _full(wl.shape),
                _full(wh.shape), _full(q.shape)],
      out_specs=[_rows(128), _PAIR],
      out_shape=[
          jax.ShapeDtypeStruct((T8, 128), jnp.float32),
          jax.ShapeDtypeStruct((_NSC, T8, 128), jnp.float32),
      ],
  )(deg8, h0p, wl, wh, q)


def _tc_mid0(T8, h0p, a, dis, wl, wh, bs, vs):
  grid = (_cdiv(T8, _R8),)
  return pl.pallas_call(
      _mid0_body,
      grid=grid,
      in_specs=[
          _rows(h0p.shape[1]), _PAIR, _rows(128),
          _full(wl.shape), _full(wh.shape),
          *[_full(b.shape) for b in bs],
          *[_full(v.shape) for v in vs],
      ],
      out_specs=[_PAIR, _PAIR],
      out_shape=[
          jax.ShapeDtypeStruct((_NSC, T8, 128), jnp.float32),
          jax.ShapeDtypeStruct((_NSC, T8, 128), jnp.float32),
      ],
  )(h0p, a, dis, wl, wh, *bs, *vs)


def _tc_mid(T8, h, a, dis, ws, bs, vs):
  grid = (_cdiv(T8, _R8),)
  return pl.pallas_call(
      _mid_body,
      grid=grid,
      in_specs=[
          _PAIR, _PAIR, _rows(128),
          *[_full(w.shape) for w in ws],
          *[_full(b.shape) for b in bs],
          *[_full(v.shape) for v in vs],
      ],
      out_specs=[_PAIR, _PAIR],
      out_shape=[
          jax.ShapeDtypeStruct((_NSC, T8, 128), jnp.float32),
          jax.ShapeDtypeStruct((_NSC, T8, 128), jnp.float32),
      ],
  )(h, a, dis, *ws, *bs, *vs)


def _tc_fin(T8, h, a, dis, ws, bs, rs, rb, sl, sh):
  grid = (_cdiv(T8, _R8),)
  return pl.pallas_call(
      _fin_body,
      grid=grid,
      in_specs=[
          _PAIR, _PAIR, _rows(128),
          *[_full(w.shape) for w in ws],
          *[_full(b.shape) for b in bs],
          *[_full(r.shape) for r in rs],
          _full(rb.shape), _full(sl.shape), _full(sh.shape),
      ],
      out_specs=[_rows(256), _rows(_P)],
      out_shape=[
          jax.ShapeDtypeStruct((T8, 256), jnp.float32),
          jax.ShapeDtypeStruct((T8, _P), jnp.float32),
      ],
  )(h, a, dis, *ws, *bs, *rs, rb, sl, sh)


def kernel(X, edge_index, conv0_w0, conv0_w1, conv0_b, conv1_w0, conv1_w1,
           conv1_b, conv2_w0, conv2_w1, conv2_b, reg_w, reg_b):
  B, W, N = X.shape
  E = edge_index.shape[2]
  T = B * N
  T8 = T // _P
  npad = _cdiv(N, 256) * 256
  eye8 = jnp.eye(_P, dtype=jnp.float32)

  def k8(w):
    return jnp.kron(eye8, w)

  def wsplit(w):
    return [k8(w[:_HH, :_HH]), k8(w[_HH:, :_HH]),
            k8(w[:_HH, _HH:]), k8(w[_HH:, _HH:])]

  h0p = X.reshape(T8, _P * W)
  q = jnp.kron(eye8, jnp.ones((1, _HH), jnp.float32))
  ones_e = jnp.ones((_EC,), jnp.float32)
  zeros_d = jnp.zeros((npad,), jnp.float32)
  zeros_f = jnp.zeros((N, _HH), jnp.float32)

  degp = _deg_kernel(B, N, E)(edge_index, ones_e, zeros_d)
  deg8 = degp[:, :N].reshape(T8, _P)
  dis, z = _tc_pre(T8, deg8, h0p, k8(conv0_w1[:, :_HH]),
                   k8(conv0_w1[:, _HH:]), q)
  prop = _prop_kernel(B, N, E)

  def _prop(zp):
    a = prop(edge_index, zp.reshape(_NSC, B, N, _HH), zeros_f)
    return a.reshape(_NSC, T8, 128)

  def bsplit(b):
    return [jnp.tile(b[:_HH], _P), jnp.tile(b[_HH:], _P)]

  a = _prop(z)
  h1, z = _tc_mid0(T8, h0p, a, dis, k8(conv0_w0[:, :_HH]),
                   k8(conv0_w0[:, _HH:]), bsplit(conv0_b),
                   wsplit(conv1_w1))
  a = _prop(z)
  h2, z = _tc_mid(T8, h1, a, dis, wsplit(conv1_w0), bsplit(conv1_b),
                  wsplit(conv2_w1))
  a = _prop(z)
  sl = k8(jnp.concatenate([jnp.eye(_HH, dtype=jnp.float32),
                           jnp.zeros((_HH, _HH), jnp.float32)], axis=1))
  sh = k8(jnp.concatenate([jnp.zeros((_HH, _HH), jnp.float32),
                           jnp.eye(_HH, dtype=jnp.float32)], axis=1))
  h3, y = _tc_fin(T8, h2, a, dis, wsplit(conv2_w0), bsplit(conv2_b),
                  [k8(reg_w[:_HH]), k8(reg_w[_HH:])], reg_b, sl, sh)
  return (y.reshape(B, N), h3.reshape(T, _H))
